# baseline (speedup 1.0000x reference)
import functools

import jax
import jax.numpy as jnp
from jax import lax
from jax.experimental import pallas as pl
from jax.experimental.pallas import tpu as pltpu
from jax.experimental.pallas import tpu_sc as plsc

T = 2048; D = 2048; NH = 16; DH = 128; QS = 512; E = 8; TOPK = 2
F = 1024; SF = 1024; EPS = 1e-05
BT = 256
NT = T // BT
NPAD = 6144
NTM = NPAD // BT
TRASH = 256

_f32 = jnp.float32
_CP = pltpu.CompilerParams(vmem_limit_bytes=112 * 1024 * 1024)


def _dot(a, b):
    return lax.dot_general(a, b, (((1,), (0,)), ((), ())),
                           preferred_element_type=_f32)


def _prologue_body(hid, res_in, wqkv, wq, cq, sq, ck, sk,
                   q_out, k_out, v_out, res_out):
    res = hid[...] + res_in[...]
    res_out[...] = res
    h = res * lax.rsqrt(jnp.mean(res * res, axis=1, keepdims=True) + EPS)
    qkv = _dot(h, wqkv[...])
    q = qkv[:, :QS]
    k = qkv[:, QS:QS + DH]
    v = qkv[:, QS + DH:]
    qn = q * lax.rsqrt(jnp.mean(q * q, axis=1, keepdims=True) + EPS)
    qp = _dot(qn, wq[...])
    parts = []
    for h_i in range(NH):
        base = h_i * DH
        parts.append(qp[:, base + DH // 2:base + DH])
        parts.append(qp[:, base:base + DH // 2])
    qswap = jnp.concatenate(parts, axis=1)
    q_out[...] = qp * cq[...] + qswap * sq[...]
    kswap = jnp.concatenate([k[:, DH // 2:], k[:, :DH // 2]], axis=1)
    k_out[...] = k * ck[...] + kswap * sk[...]
    v_out[...] = v


def _prologue(hid, resid, wqkv_f, wq_f, cq, sq, ck, sk):
    return pl.pallas_call(
        _prologue_body,
        grid=(NT,),
        in_specs=[
            pl.BlockSpec((BT, D), lambda i: (i, 0)),
            pl.BlockSpec((BT, D), lambda i: (i, 0)),
            pl.BlockSpec((D, QS + 2 * DH), lambda i: (0, 0)),
            pl.BlockSpec((QS, NH * DH), lambda i: (0, 0)),
            pl.BlockSpec((BT, NH * DH), lambda i: (i, 0)),
            pl.BlockSpec((BT, NH * DH), lambda i: (i, 0)),
            pl.BlockSpec((BT, DH), lambda i: (i, 0)),
            pl.BlockSpec((BT, DH), lambda i: (i, 0)),
        ],
        out_specs=[
            pl.BlockSpec((BT, NH * DH), lambda i: (i, 0)),
            pl.BlockSpec((BT, DH), lambda i: (i, 0)),
            pl.BlockSpec((BT, DH), lambda i: (i, 0)),
            pl.BlockSpec((BT, D), lambda i: (i, 0)),
        ],
        out_shape=[
            jax.ShapeDtypeStruct((T, NH * DH), _f32),
            jax.ShapeDtypeStruct((T, DH), _f32),
            jax.ShapeDtypeStruct((T, DH), _f32),
            jax.ShapeDtypeStruct((T, D), _f32),
        ],
    )(hid, resid, wqkv_f, wq_f, cq, sq, ck, sk)


def _attn_body(q_ref, k_ref, v_ref, o_ref):
    i = pl.program_id(0)
    row = i * BT + lax.broadcasted_iota(jnp.int32, (BT, T), 0)
    col = lax.broadcasted_iota(jnp.int32, (BT, T), 1)
    neg = jnp.float32(-1e30)
    k = k_ref[...]
    v = v_ref[...]
    outs = []
    for h_i in range(NH):
        qh = q_ref[:, h_i * DH:(h_i + 1) * DH]
        s = lax.dot_general(qh, k, (((1,), (1,)), ((), ())),
                            preferred_element_type=_f32) * (DH ** -0.5)
        s = jnp.where(col <= row, s, neg)
        m = jnp.max(s, axis=1, keepdims=True)
        p = jnp.exp(s - m)
        l = jnp.sum(p, axis=1, keepdims=True)
        outs.append(_dot(p / l, v))
    o_ref[...] = jnp.concatenate(outs, axis=1)


def _attention(q, k, v):
    return pl.pallas_call(
        _attn_body,
        grid=(NT,),
        in_specs=[
            pl.BlockSpec((BT, NH * DH), lambda i: (i, 0)),
            pl.BlockSpec((T, DH), lambda i: (0, 0)),
            pl.BlockSpec((T, DH), lambda i: (0, 0)),
        ],
        out_specs=pl.BlockSpec((BT, NH * DH), lambda i: (i, 0)),
        out_shape=jax.ShapeDtypeStruct((T, NH * DH), _f32),
    )(q, k, v)


def _post_body(ao, res, wo, wlnp, wsgu, wsd, wg,
               res2_out, h2_out, share_out, pv_out, pi_out):
    attn = _dot(ao[...], wo[...])
    res2 = attn + res[...]
    res2_out[...] = res2
    h2 = res2 * lax.rsqrt(jnp.mean(res2 * res2, axis=1, keepdims=True) + EPS)
    h2 = h2 * wlnp[...]
    h2_out[...] = h2
    sgu = _dot(h2, wsgu[...])
    g1 = sgu[:, :SF]
    g2 = sgu[:, SF:]
    act = g1 * jax.nn.sigmoid(g1) * g2
    share_out[...] = _dot(act, wsd[...])
    logits = _dot(h2, wg[...])
    lane = lax.broadcasted_iota(jnp.int32, (BT, 128), 1)
    logits = jnp.where(lane < E, logits, jnp.float32(-1e30))
    m = jnp.max(logits, axis=1, keepdims=True)
    ex = jnp.exp(logits - m)
    probs = ex / jnp.sum(ex, axis=1, keepdims=True)
    m1 = jnp.max(probs, axis=1, keepdims=True)
    i1 = jnp.min(jnp.where(probs == m1, lane, 128), axis=1, keepdims=True)
    p2 = jnp.where(lane == i1, jnp.float32(-1.0), probs)
    m2 = jnp.max(p2, axis=1, keepdims=True)
    i2 = jnp.min(jnp.where(p2 == m2, lane, 128), axis=1, keepdims=True)
    wsum = m1 + m2
    w1 = m1 / wsum
    w2 = m2 / wsum
    pv_out[...] = jnp.where(lane == 0, w1, jnp.where(lane == 1, w2, 0.0))
    pi_out[...] = jnp.where(lane == 0, i1, jnp.where(lane == 1, i2, 0))


def _post(ao, res, w_o, wlnp2d, w_share_gu, w_share_down, wg_pad):
    return pl.pallas_call(
        _post_body,
        grid=(NT,),
        in_specs=[
            pl.BlockSpec((BT, NH * DH), lambda i: (i, 0)),
            pl.BlockSpec((BT, D), lambda i: (i, 0)),
            pl.BlockSpec((NH * DH, D), lambda i: (0, 0)),
            pl.BlockSpec((1, D), lambda i: (0, 0)),
            pl.BlockSpec((D, 2 * SF), lambda i: (0, 0)),
            pl.BlockSpec((SF, D), lambda i: (0, 0)),
            pl.BlockSpec((D, 128), lambda i: (0, 0)),
        ],
        out_specs=[
            pl.BlockSpec((BT, D), lambda i: (i, 0)),
            pl.BlockSpec((BT, D), lambda i: (i, 0)),
            pl.BlockSpec((BT, D), lambda i: (i, 0)),
            pl.BlockSpec((BT, 128), lambda i: (i, 0)),
            pl.BlockSpec((BT, 128), lambda i: (i, 0)),
        ],
        out_shape=[
            jax.ShapeDtypeStruct((T, D), _f32),
            jax.ShapeDtypeStruct((T, D), _f32),
            jax.ShapeDtypeStruct((T, D), _f32),
            jax.ShapeDtypeStruct((T, 128), _f32),
            jax.ShapeDtypeStruct((T, 128), jnp.int32),
        ],
        compiler_params=_CP,
    )(ao, res, w_o, wlnp2d, w_share_gu, w_share_down, wg_pad)


def _moe_body(eid_ref, xg, ws, wgu, wdn, eo):
    del eid_ref
    g = _dot(xg[...], wgu[0])
    g1 = g[:, :F]
    g2 = g[:, F:]
    act = g1 * jax.nn.sigmoid(g1) * g2
    act = act * ws[:, :1]
    eo[...] = _dot(act, wdn[0])


def _moe(eid, xg, ws2d, w_gu, w_down):
    grid_spec = pltpu.PrefetchScalarGridSpec(
        num_scalar_prefetch=1,
        grid=(NTM,),
        in_specs=[
            pl.BlockSpec((BT, D), lambda i, eid: (i, 0)),
            pl.BlockSpec((BT, 128), lambda i, eid: (i, 0)),
            pl.BlockSpec((1, D, 2 * F), lambda i, eid: (eid[i], 0, 0)),
            pl.BlockSpec((1, F, D), lambda i, eid: (eid[i], 0, 0)),
        ],
        out_specs=pl.BlockSpec((BT, D), lambda i, eid: (i, 0)),
    )
    return pl.pallas_call(
        _moe_body,
        grid_spec=grid_spec,
        out_shape=jax.ShapeDtypeStruct((NPAD, D), _f32),
        compiler_params=_CP,
    )(eid, xg, ws2d, w_gu, w_down)


_NC = 2
_NS = 16
_NW = _NC * _NS
_ROWS_W = NPAD // _NW
_CH = 24
_NCH = _ROWS_W // _CH


def _sc_mesh():
    return plsc.VectorSubcoreMesh(core_axis_name="c", subcore_axis_name="s")


def _sc_scratch():
    return [
        pltpu.VMEM((_NCH, _CH), jnp.int32),
        pltpu.VMEM((2, _CH, D), _f32),
        pltpu.SemaphoreType.DMA,
        pltpu.SemaphoreType.DMA,
        pltpu.SemaphoreType.DMA,
        pltpu.SemaphoreType.DMA,
    ]


def _sc_pipeline(idx2d_hbm, wid, read_chunk, write_chunk, idx_v, rows_v,
                 gsems, wsems):
    pltpu.sync_copy(idx2d_hbm.at[pl.ds(wid * _NCH, _NCH)], idx_v)
    gat = [None] * _NCH
    wrt = [None] * _NCH
    gat[0] = read_chunk(0, rows_v.at[0], gsems[0])
    for c in range(_NCH):
        if c + 1 < _NCH:
            if c - 1 >= 0:
                wrt[c - 1].wait()
            gat[c + 1] = read_chunk(c + 1, rows_v.at[(c + 1) % 2],
                                    gsems[(c + 1) % 2])
        gat[c].wait()
        wrt[c] = write_chunk(c, rows_v.at[c % 2], wsems[c % 2])
    wrt[_NCH - 2].wait()
    wrt[_NCH - 1].wait()


def _dispatch_body(h2_hbm, idx_hbm, out_hbm, idx_v, rows_v, s0, s1, s2, s3):
    wid = lax.axis_index("s") * _NC + lax.axis_index("c")
    base = wid * _ROWS_W

    def read_chunk(c, buf, sem):
        return pltpu.async_copy(h2_hbm.at[idx_v.at[c]], buf, sem)

    def write_chunk(c, buf, sem):
        return pltpu.async_copy(
            buf, out_hbm.at[pl.ds(base + c * _CH, _CH)], sem)

    _sc_pipeline(idx_hbm, wid, read_chunk, write_chunk, idx_v, rows_v,
                 (s0, s1), (s2, s3))


def _dispatch(h2, row_ids2d):
    return pl.kernel(
        _dispatch_body,
        out_type=jax.ShapeDtypeStruct((NPAD, D), _f32),
        mesh=_sc_mesh(),
        scratch_types=_sc_scratch(),
    )(h2, row_ids2d)


def _combine_body(eo_hbm, tgt_hbm, out_hbm, idx_v, rows_v, s0, s1, s2, s3):
    wid = lax.axis_index("s") * _NC + lax.axis_index("c")
    base = wid * _ROWS_W

    def read_chunk(c, buf, sem):
        return pltpu.async_copy(
            eo_hbm.at[pl.ds(base + c * _CH, _CH)], buf, sem)

    def write_chunk(c, buf, sem):
        return pltpu.async_copy(buf, out_hbm.at[idx_v.at[c]], sem)

    _sc_pipeline(tgt_hbm, wid, read_chunk, write_chunk, idx_v, rows_v,
                 (s0, s1), (s2, s3))


def _combine(eo, tgt2d):
    return pl.kernel(
        _combine_body,
        out_type=jax.ShapeDtypeStruct((2 * T + TRASH, D), _f32),
        mesh=_sc_mesh(),
        scratch_types=_sc_scratch(),
    )(eo, tgt2d)


def _final_body(share, b0, b1, out):
    out[...] = share[...] + b0[...] + b1[...]


def _final(share, buf):
    return pl.pallas_call(
        _final_body,
        grid=(NT,),
        in_specs=[
            pl.BlockSpec((BT, D), lambda i: (i, 0)),
            pl.BlockSpec((BT, D), lambda i: (i, 0)),
            pl.BlockSpec((BT, D), lambda i: (i + NT, 0)),
        ],
        out_specs=pl.BlockSpec((BT, D), lambda i: (i, 0)),
        out_shape=jax.ShapeDtypeStruct((T, D), _f32),
    )(share, buf, buf)


def kernel(positions, hidden_states, residual, w_ln_in, w_qkv, w_inter,
           w_q, w_o, w_ln_post, w_gate, w_gu_experts, w_down_experts,
           w_share_gu, w_share_down):
    wqkv_f = w_ln_in[:, None] * w_qkv
    wq_f = w_inter[:, None] * w_q
    wlnp2d = w_ln_post.reshape(1, D)
    wg_pad = jnp.zeros((D, 128), _f32).at[:, :E].set(w_gate)
    inv = 1.0 / (10000.0 ** (jnp.arange(0, DH, 2, dtype=_f32) / DH))
    f = positions.astype(_f32)[:, None] * inv[None, :]
    cos = jnp.cos(f)
    sin = jnp.sin(f)
    ck = jnp.concatenate([cos, cos], axis=1)
    sk = jnp.concatenate([-sin, sin], axis=1)
    cq = jnp.tile(ck, (1, NH))
    sq = jnp.tile(sk, (1, NH))

    q, k, v, res = _prologue(hidden_states, residual, wqkv_f, wq_f,
                             cq, sq, ck, sk)
    ao = _attention(q, k, v)
    res2, h2, share, pv, pi = _post(ao, res, w_o, wlnp2d,
                                    w_share_gu, w_share_down, wg_pad)

    i32 = jnp.int32
    e_flat = jnp.stack([pi[:, 0], pi[:, 1]], axis=1).reshape(-1)
    w_flat = jnp.stack([pv[:, 0], pv[:, 1]], axis=1).reshape(-1)
    tok = jnp.arange(T, dtype=i32)
    tok_flat = jnp.stack([tok, tok], axis=1).reshape(-1)
    tgt_tok = jnp.stack([tok, tok + T], axis=1).reshape(-1)
    oh = (e_flat[:, None] == jnp.arange(E, dtype=i32)[None, :]).astype(i32)
    counts = jnp.sum(oh, axis=0)
    rank = jnp.sum((jnp.cumsum(oh, axis=0) - oh) * oh, axis=1)
    cap = ((counts + BT - 1) // BT) * BT
    cum = jnp.cumsum(cap)
    off = cum - cap
    pos = off[e_flat] + rank
    row_ids = jnp.zeros((NPAD,), i32).at[pos].set(tok_flat)
    ws = jnp.zeros((NPAD,), _f32).at[pos].set(w_flat)
    tgt = (2 * T + (jnp.arange(NPAD, dtype=i32) % TRASH)).at[pos].set(tgt_tok)
    eid = jnp.minimum(
        jnp.sum(jnp.arange(NTM, dtype=i32)[:, None] * BT >= cum[None, :],
                axis=1), E - 1).astype(i32)

    xg = _dispatch(h2, row_ids.reshape(NPAD // _CH, _CH))
    ws2d = jnp.broadcast_to(ws[:, None], (NPAD, 128))
    eo = _moe(eid, xg, ws2d, w_gu_experts, w_down_experts)
    buf = _combine(eo, tgt.reshape(NPAD // _CH, _CH))
    out = _final(share, buf)
    return out, res2


# speedup vs baseline: 1.0069x; 1.0069x over previous
import functools

import jax
import jax.numpy as jnp
from jax import lax
from jax.experimental import pallas as pl
from jax.experimental.pallas import tpu as pltpu
from jax.experimental.pallas import tpu_sc as plsc

T = 2048; D = 2048; NH = 16; DH = 128; QS = 512; E = 8; TOPK = 2
F = 1024; SF = 1024; EPS = 1e-05
BT = 256
NT = T // BT
NPAD = 6144
NTM = NPAD // BT
TRASH = NPAD

_f32 = jnp.float32
_CP = pltpu.CompilerParams(vmem_limit_bytes=112 * 1024 * 1024)


def _dot(a, b):
    return lax.dot_general(a, b, (((1,), (0,)), ((), ())),
                           preferred_element_type=_f32)


def _prologue_body(hid, res_in, wqkv, wq, cq, sq, ck, sk,
                   q_out, k_out, v_out, res_out):
    res = hid[...] + res_in[...]
    res_out[...] = res
    h = res * lax.rsqrt(jnp.mean(res * res, axis=1, keepdims=True) + EPS)
    qkv = _dot(h, wqkv[...])
    q = qkv[:, :QS]
    k = qkv[:, QS:QS + DH]
    v = qkv[:, QS + DH:]
    qn = q * lax.rsqrt(jnp.mean(q * q, axis=1, keepdims=True) + EPS)
    qp = _dot(qn, wq[...])
    parts = []
    for h_i in range(NH):
        base = h_i * DH
        parts.append(qp[:, base + DH // 2:base + DH])
        parts.append(qp[:, base:base + DH // 2])
    qswap = jnp.concatenate(parts, axis=1)
    q_out[...] = qp * cq[...] + qswap * sq[...]
    kswap = jnp.concatenate([k[:, DH // 2:], k[:, :DH // 2]], axis=1)
    k_out[...] = k * ck[...] + kswap * sk[...]
    v_out[...] = v


def _prologue(hid, resid, wqkv_f, wq_f, cq, sq, ck, sk):
    return pl.pallas_call(
        _prologue_body,
        grid=(NT,),
        in_specs=[
            pl.BlockSpec((BT, D), lambda i: (i, 0)),
            pl.BlockSpec((BT, D), lambda i: (i, 0)),
            pl.BlockSpec((D, QS + 2 * DH), lambda i: (0, 0)),
            pl.BlockSpec((QS, NH * DH), lambda i: (0, 0)),
            pl.BlockSpec((BT, NH * DH), lambda i: (i, 0)),
            pl.BlockSpec((BT, NH * DH), lambda i: (i, 0)),
            pl.BlockSpec((BT, DH), lambda i: (i, 0)),
            pl.BlockSpec((BT, DH), lambda i: (i, 0)),
        ],
        out_specs=[
            pl.BlockSpec((BT, NH * DH), lambda i: (i, 0)),
            pl.BlockSpec((BT, DH), lambda i: (i, 0)),
            pl.BlockSpec((BT, DH), lambda i: (i, 0)),
            pl.BlockSpec((BT, D), lambda i: (i, 0)),
        ],
        out_shape=[
            jax.ShapeDtypeStruct((T, NH * DH), _f32),
            jax.ShapeDtypeStruct((T, DH), _f32),
            jax.ShapeDtypeStruct((T, DH), _f32),
            jax.ShapeDtypeStruct((T, D), _f32),
        ],
    )(hid, resid, wqkv_f, wq_f, cq, sq, ck, sk)


def _attn_body(q_ref, k_ref, v_ref, o_ref):
    i = pl.program_id(0)
    row = i * BT + lax.broadcasted_iota(jnp.int32, (BT, T), 0)
    col = lax.broadcasted_iota(jnp.int32, (BT, T), 1)
    neg = jnp.float32(-1e30)
    k = k_ref[...]
    v = v_ref[...]
    outs = []
    for h_i in range(NH):
        qh = q_ref[:, h_i * DH:(h_i + 1) * DH]
        s = lax.dot_general(qh, k, (((1,), (1,)), ((), ())),
                            preferred_element_type=_f32) * (DH ** -0.5)
        s = jnp.where(col <= row, s, neg)
        m = jnp.max(s, axis=1, keepdims=True)
        p = jnp.exp(s - m)
        l = jnp.sum(p, axis=1, keepdims=True)
        outs.append(_dot(p / l, v))
    o_ref[...] = jnp.concatenate(outs, axis=1)


def _attention(q, k, v):
    return pl.pallas_call(
        _attn_body,
        grid=(NT,),
        in_specs=[
            pl.BlockSpec((BT, NH * DH), lambda i: (i, 0)),
            pl.BlockSpec((T, DH), lambda i: (0, 0)),
            pl.BlockSpec((T, DH), lambda i: (0, 0)),
        ],
        out_specs=pl.BlockSpec((BT, NH * DH), lambda i: (i, 0)),
        out_shape=jax.ShapeDtypeStruct((T, NH * DH), _f32),
    )(q, k, v)


def _post_body(ao, res, wo, wlnp, wsgu, wsd, wg,
               res2_out, h2_out, share_out, pv_out, pi_out):
    attn = _dot(ao[...], wo[...])
    res2 = attn + res[...]
    res2_out[...] = res2
    h2 = res2 * lax.rsqrt(jnp.mean(res2 * res2, axis=1, keepdims=True) + EPS)
    h2 = h2 * wlnp[...]
    h2_out[...] = h2
    sgu = _dot(h2, wsgu[...])
    g1 = sgu[:, :SF]
    g2 = sgu[:, SF:]
    act = g1 * jax.nn.sigmoid(g1) * g2
    share_out[...] = _dot(act, wsd[...])
    logits = _dot(h2, wg[...])
    lane = lax.broadcasted_iota(jnp.int32, (BT, 128), 1)
    logits = jnp.where(lane < E, logits, jnp.float32(-1e30))
    m = jnp.max(logits, axis=1, keepdims=True)
    ex = jnp.exp(logits - m)
    probs = ex / jnp.sum(ex, axis=1, keepdims=True)
    m1 = jnp.max(probs, axis=1, keepdims=True)
    i1 = jnp.min(jnp.where(probs == m1, lane, 128), axis=1, keepdims=True)
    p2 = jnp.where(lane == i1, jnp.float32(-1.0), probs)
    m2 = jnp.max(p2, axis=1, keepdims=True)
    i2 = jnp.min(jnp.where(p2 == m2, lane, 128), axis=1, keepdims=True)
    wsum = m1 + m2
    w1 = m1 / wsum
    w2 = m2 / wsum
    pv_out[...] = jnp.where(lane == 0, w1, jnp.where(lane == 1, w2, 0.0))
    pi_out[...] = jnp.where(lane == 0, i1, jnp.where(lane == 1, i2, 0))


def _post(ao, res, w_o, wlnp2d, w_share_gu, w_share_down, wg_pad):
    return pl.pallas_call(
        _post_body,
        grid=(NT,),
        in_specs=[
            pl.BlockSpec((BT, NH * DH), lambda i: (i, 0)),
            pl.BlockSpec((BT, D), lambda i: (i, 0)),
            pl.BlockSpec((NH * DH, D), lambda i: (0, 0)),
            pl.BlockSpec((1, D), lambda i: (0, 0)),
            pl.BlockSpec((D, 2 * SF), lambda i: (0, 0)),
            pl.BlockSpec((SF, D), lambda i: (0, 0)),
            pl.BlockSpec((D, 128), lambda i: (0, 0)),
        ],
        out_specs=[
            pl.BlockSpec((BT, D), lambda i: (i, 0)),
            pl.BlockSpec((BT, D), lambda i: (i, 0)),
            pl.BlockSpec((BT, D), lambda i: (i, 0)),
            pl.BlockSpec((BT, 128), lambda i: (i, 0)),
            pl.BlockSpec((BT, 128), lambda i: (i, 0)),
        ],
        out_shape=[
            jax.ShapeDtypeStruct((T, D), _f32),
            jax.ShapeDtypeStruct((T, D), _f32),
            jax.ShapeDtypeStruct((T, D), _f32),
            jax.ShapeDtypeStruct((T, 128), _f32),
            jax.ShapeDtypeStruct((T, 128), jnp.int32),
        ],
        compiler_params=_CP,
    )(ao, res, w_o, wlnp2d, w_share_gu, w_share_down, wg_pad)


def _moe_body(eid_ref, xg, ws, wgu, wdn, eo):
    del eid_ref
    g = _dot(xg[...], wgu[0])
    g1 = g[:, :F]
    g2 = g[:, F:]
    act = g1 * jax.nn.sigmoid(g1) * g2
    act = act * ws[:, :1]
    eo[...] = _dot(act, wdn[0])


def _moe(eid, xg, ws2d, w_gu, w_down):
    grid_spec = pltpu.PrefetchScalarGridSpec(
        num_scalar_prefetch=1,
        grid=(NTM,),
        in_specs=[
            pl.BlockSpec((BT, D), lambda i, eid: (i, 0)),
            pl.BlockSpec((BT, 128), lambda i, eid: (i, 0)),
            pl.BlockSpec((1, D, 2 * F), lambda i, eid: (eid[i], 0, 0)),
            pl.BlockSpec((1, F, D), lambda i, eid: (eid[i], 0, 0)),
        ],
        out_specs=pl.BlockSpec((BT, D), lambda i, eid: (i, 0)),
    )
    return pl.pallas_call(
        _moe_body,
        grid_spec=grid_spec,
        out_shape=jax.ShapeDtypeStruct((NPAD, D), _f32),
        compiler_params=_CP,
    )(eid, xg, ws2d, w_gu, w_down)


_NC = 2
_NS = 16
_NW = _NC * _NS
_ROWS_W = NPAD // _NW
_CH = 24
_NCH = _ROWS_W // _CH


def _sc_mesh():
    return plsc.VectorSubcoreMesh(core_axis_name="c", subcore_axis_name="s")


def _sc_scratch():
    return [
        pltpu.VMEM((_NCH, _CH), jnp.int32),
        pltpu.VMEM((2, _CH, D), _f32),
        pltpu.SemaphoreType.DMA,
        pltpu.SemaphoreType.DMA,
        pltpu.SemaphoreType.DMA,
        pltpu.SemaphoreType.DMA,
    ]


def _sc_pipeline(idx2d_hbm, wid, read_chunk, write_chunk, idx_v, rows_v,
                 gsems, wsems):
    pltpu.sync_copy(idx2d_hbm.at[pl.ds(wid * _NCH, _NCH)], idx_v)
    gat = [None] * _NCH
    wrt = [None] * _NCH
    gat[0] = read_chunk(0, rows_v.at[0], gsems[0])
    for c in range(_NCH):
        if c + 1 < _NCH:
            if c - 1 >= 0:
                wrt[c - 1].wait()
            gat[c + 1] = read_chunk(c + 1, rows_v.at[(c + 1) % 2],
                                    gsems[(c + 1) % 2])
        gat[c].wait()
        wrt[c] = write_chunk(c, rows_v.at[c % 2], wsems[c % 2])
    wrt[_NCH - 2].wait()
    wrt[_NCH - 1].wait()


def _dispatch_body(h2_hbm, idx_hbm, out_hbm, idx_v, rows_v, s0, s1, s2, s3):
    wid = lax.axis_index("s") * _NC + lax.axis_index("c")
    base = wid * _ROWS_W

    def read_chunk(c, buf, sem):
        return pltpu.async_copy(h2_hbm.at[idx_v.at[c]], buf, sem)

    def write_chunk(c, buf, sem):
        return pltpu.async_copy(
            buf, out_hbm.at[pl.ds(base + c * _CH, _CH)], sem)

    _sc_pipeline(idx_hbm, wid, read_chunk, write_chunk, idx_v, rows_v,
                 (s0, s1), (s2, s3))


def _dispatch(h2, row_ids2d):
    return pl.kernel(
        _dispatch_body,
        out_type=jax.ShapeDtypeStruct((NPAD, D), _f32),
        mesh=_sc_mesh(),
        scratch_types=_sc_scratch(),
    )(h2, row_ids2d)


def _combine_body(eo_hbm, tgt_hbm, out_hbm, idx_v, rows_v, s0, s1, s2, s3):
    wid = lax.axis_index("s") * _NC + lax.axis_index("c")
    base = wid * _ROWS_W

    def read_chunk(c, buf, sem):
        return pltpu.async_copy(
            eo_hbm.at[pl.ds(base + c * _CH, _CH)], buf, sem)

    def write_chunk(c, buf, sem):
        return pltpu.async_copy(buf, out_hbm.at[idx_v.at[c]], sem)

    _sc_pipeline(tgt_hbm, wid, read_chunk, write_chunk, idx_v, rows_v,
                 (s0, s1), (s2, s3))


def _combine(eo, tgt2d):
    return pl.kernel(
        _combine_body,
        out_type=jax.ShapeDtypeStruct((2 * T + TRASH, D), _f32),
        mesh=_sc_mesh(),
        scratch_types=_sc_scratch(),
    )(eo, tgt2d)


def _final_body(share, b0, b1, out):
    out[...] = share[...] + b0[...] + b1[...]


def _final(share, buf):
    return pl.pallas_call(
        _final_body,
        grid=(NT,),
        in_specs=[
            pl.BlockSpec((BT, D), lambda i: (i, 0)),
            pl.BlockSpec((BT, D), lambda i: (i, 0)),
            pl.BlockSpec((BT, D), lambda i: (i + NT, 0)),
        ],
        out_specs=pl.BlockSpec((BT, D), lambda i: (i, 0)),
        out_shape=jax.ShapeDtypeStruct((T, D), _f32),
    )(share, buf, buf)


def kernel(positions, hidden_states, residual, w_ln_in, w_qkv, w_inter,
           w_q, w_o, w_ln_post, w_gate, w_gu_experts, w_down_experts,
           w_share_gu, w_share_down):
    wqkv_f = w_ln_in[:, None] * w_qkv
    wq_f = w_inter[:, None] * w_q
    wlnp2d = w_ln_post.reshape(1, D)
    wg_pad = jnp.zeros((D, 128), _f32).at[:, :E].set(w_gate)
    inv = 1.0 / (10000.0 ** (jnp.arange(0, DH, 2, dtype=_f32) / DH))
    f = positions.astype(_f32)[:, None] * inv[None, :]
    cos = jnp.cos(f)
    sin = jnp.sin(f)
    ck = jnp.concatenate([cos, cos], axis=1)
    sk = jnp.concatenate([-sin, sin], axis=1)
    cq = jnp.tile(ck, (1, NH))
    sq = jnp.tile(sk, (1, NH))

    q, k, v, res = _prologue(hidden_states, residual, wqkv_f, wq_f,
                             cq, sq, ck, sk)
    ao = _attention(q, k, v)
    res2, h2, share, pv, pi = _post(ao, res, w_o, wlnp2d,
                                    w_share_gu, w_share_down, wg_pad)

    i32 = jnp.int32
    e_flat = jnp.stack([pi[:, 0], pi[:, 1]], axis=1).reshape(-1)
    w_flat = jnp.stack([pv[:, 0], pv[:, 1]], axis=1).reshape(-1)
    tok = jnp.arange(T, dtype=i32)
    tok_flat = jnp.stack([tok, tok], axis=1).reshape(-1)
    tgt_tok = jnp.stack([tok, tok + T], axis=1).reshape(-1)
    oh = (e_flat[:, None] == jnp.arange(E, dtype=i32)[None, :]).astype(i32)
    counts = jnp.sum(oh, axis=0)
    rank = jnp.sum((jnp.cumsum(oh, axis=0) - oh) * oh, axis=1)
    cap = ((counts + BT - 1) // BT) * BT
    cum = jnp.cumsum(cap)
    off = cum - cap
    pos = off[e_flat] + rank
    row_ids = jnp.zeros((NPAD,), i32).at[pos].set(tok_flat)
    ws = jnp.zeros((NPAD,), _f32).at[pos].set(w_flat)
    tgt = (2 * T + jnp.arange(NPAD, dtype=i32)).at[pos].set(tgt_tok)
    eid = jnp.minimum(
        jnp.sum(jnp.arange(NTM, dtype=i32)[:, None] * BT >= cum[None, :],
                axis=1), E - 1).astype(i32)

    xg = _dispatch(h2, row_ids.reshape(NPAD // _CH, _CH))
    ws2d = jnp.broadcast_to(ws[:, None], (NPAD, 128))
    eo = _moe(eid, xg, ws2d, w_gu_experts, w_down_experts)
    buf = _combine(eo, tgt.reshape(NPAD // _CH, _CH))
    out = _final(share, buf)
    return out, res2


# speedup vs baseline: 1.3236x; 1.3145x over previous
import functools

import jax
import jax.numpy as jnp
from jax import lax
from jax.experimental import pallas as pl
from jax.experimental.pallas import tpu as pltpu
from jax.experimental.pallas import tpu_sc as plsc

T = 2048; D = 2048; NH = 16; DH = 128; QS = 512; E = 8; TOPK = 2
F = 1024; SF = 1024; EPS = 1e-05
BT = 256
NT = T // BT
NPAD = 6144
NTM = NPAD // BT
TRASH = NPAD

_f32 = jnp.float32
_CP = pltpu.CompilerParams(vmem_limit_bytes=112 * 1024 * 1024)


def _dot(a, b):
    return lax.dot_general(a, b, (((1,), (0,)), ((), ())),
                           preferred_element_type=_f32)


def _prologue_body(hid, res_in, wqkv, wq, cq, sq, ck, sk,
                   q_out, k_out, v_out, res_out):
    res = hid[...] + res_in[...]
    res_out[...] = res
    h = res * lax.rsqrt(jnp.mean(res * res, axis=1, keepdims=True) + EPS)
    qkv = _dot(h, wqkv[...])
    q = qkv[:, :QS]
    k = qkv[:, QS:QS + DH]
    v = qkv[:, QS + DH:]
    qn = q * lax.rsqrt(jnp.mean(q * q, axis=1, keepdims=True) + EPS)
    qp = _dot(qn, wq[...])
    parts = []
    for h_i in range(NH):
        base = h_i * DH
        parts.append(qp[:, base + DH // 2:base + DH])
        parts.append(qp[:, base:base + DH // 2])
    qswap = jnp.concatenate(parts, axis=1)
    q_out[...] = qp * cq[...] + qswap * sq[...]
    kswap = jnp.concatenate([k[:, DH // 2:], k[:, :DH // 2]], axis=1)
    k_out[...] = k * ck[...] + kswap * sk[...]
    v_out[...] = v


def _prologue(hid, resid, wqkv_f, wq_f, cq, sq, ck, sk):
    return pl.pallas_call(
        _prologue_body,
        grid=(NT,),
        in_specs=[
            pl.BlockSpec((BT, D), lambda i: (i, 0)),
            pl.BlockSpec((BT, D), lambda i: (i, 0)),
            pl.BlockSpec((D, QS + 2 * DH), lambda i: (0, 0)),
            pl.BlockSpec((QS, NH * DH), lambda i: (0, 0)),
            pl.BlockSpec((BT, NH * DH), lambda i: (i, 0)),
            pl.BlockSpec((BT, NH * DH), lambda i: (i, 0)),
            pl.BlockSpec((BT, DH), lambda i: (i, 0)),
            pl.BlockSpec((BT, DH), lambda i: (i, 0)),
        ],
        out_specs=[
            pl.BlockSpec((BT, NH * DH), lambda i: (i, 0)),
            pl.BlockSpec((BT, DH), lambda i: (i, 0)),
            pl.BlockSpec((BT, DH), lambda i: (i, 0)),
            pl.BlockSpec((BT, D), lambda i: (i, 0)),
        ],
        out_shape=[
            jax.ShapeDtypeStruct((T, NH * DH), _f32),
            jax.ShapeDtypeStruct((T, DH), _f32),
            jax.ShapeDtypeStruct((T, DH), _f32),
            jax.ShapeDtypeStruct((T, D), _f32),
        ],
    )(hid, resid, wqkv_f, wq_f, cq, sq, ck, sk)


def _attn_body(q_ref, k_ref, v_ref, o_ref):
    i = pl.program_id(0)
    row = i * BT + lax.broadcasted_iota(jnp.int32, (BT, T), 0)
    col = lax.broadcasted_iota(jnp.int32, (BT, T), 1)
    neg = jnp.float32(-1e30)
    k = k_ref[...]
    v = v_ref[...]
    outs = []
    for h_i in range(NH):
        qh = q_ref[:, h_i * DH:(h_i + 1) * DH]
        s = lax.dot_general(qh, k, (((1,), (1,)), ((), ())),
                            preferred_element_type=_f32) * (DH ** -0.5)
        s = jnp.where(col <= row, s, neg)
        m = jnp.max(s, axis=1, keepdims=True)
        p = jnp.exp(s - m)
        l = jnp.sum(p, axis=1, keepdims=True)
        outs.append(_dot(p / l, v))
    o_ref[...] = jnp.concatenate(outs, axis=1)


def _attention(q, k, v):
    return pl.pallas_call(
        _attn_body,
        grid=(NT,),
        in_specs=[
            pl.BlockSpec((BT, NH * DH), lambda i: (i, 0)),
            pl.BlockSpec((T, DH), lambda i: (0, 0)),
            pl.BlockSpec((T, DH), lambda i: (0, 0)),
        ],
        out_specs=pl.BlockSpec((BT, NH * DH), lambda i: (i, 0)),
        out_shape=jax.ShapeDtypeStruct((T, NH * DH), _f32),
    )(q, k, v)


def _post_body(ao, res, wo, wlnp, wsgu, wsd, wg,
               res2_out, h2_out, share_out, pv_out, pi_out):
    attn = _dot(ao[...], wo[...])
    res2 = attn + res[...]
    res2_out[...] = res2
    h2 = res2 * lax.rsqrt(jnp.mean(res2 * res2, axis=1, keepdims=True) + EPS)
    h2 = h2 * wlnp[...]
    h2_out[...] = h2
    sgu = _dot(h2, wsgu[...])
    g1 = sgu[:, :SF]
    g2 = sgu[:, SF:]
    act = g1 * jax.nn.sigmoid(g1) * g2
    share_out[...] = _dot(act, wsd[...])
    logits = _dot(h2, wg[...])
    lane = lax.broadcasted_iota(jnp.int32, (BT, 128), 1)
    logits = jnp.where(lane < E, logits, jnp.float32(-1e30))
    m = jnp.max(logits, axis=1, keepdims=True)
    ex = jnp.exp(logits - m)
    probs = ex / jnp.sum(ex, axis=1, keepdims=True)
    m1 = jnp.max(probs, axis=1, keepdims=True)
    i1 = jnp.min(jnp.where(probs == m1, lane, 128), axis=1, keepdims=True)
    p2 = jnp.where(lane == i1, jnp.float32(-1.0), probs)
    m2 = jnp.max(p2, axis=1, keepdims=True)
    i2 = jnp.min(jnp.where(p2 == m2, lane, 128), axis=1, keepdims=True)
    wsum = m1 + m2
    w1 = m1 / wsum
    w2 = m2 / wsum
    pv_out[...] = jnp.where(lane == 0, w1, jnp.where(lane == 1, w2, 0.0))
    pi_out[...] = jnp.where(lane == 0, i1, jnp.where(lane == 1, i2, 0))


def _post(ao, res, w_o, wlnp2d, w_share_gu, w_share_down, wg_pad):
    return pl.pallas_call(
        _post_body,
        grid=(NT,),
        in_specs=[
            pl.BlockSpec((BT, NH * DH), lambda i: (i, 0)),
            pl.BlockSpec((BT, D), lambda i: (i, 0)),
            pl.BlockSpec((NH * DH, D), lambda i: (0, 0)),
            pl.BlockSpec((1, D), lambda i: (0, 0)),
            pl.BlockSpec((D, 2 * SF), lambda i: (0, 0)),
            pl.BlockSpec((SF, D), lambda i: (0, 0)),
            pl.BlockSpec((D, 128), lambda i: (0, 0)),
        ],
        out_specs=[
            pl.BlockSpec((BT, D), lambda i: (i, 0)),
            pl.BlockSpec((BT, D), lambda i: (i, 0)),
            pl.BlockSpec((BT, D), lambda i: (i, 0)),
            pl.BlockSpec((BT, 128), lambda i: (i, 0)),
            pl.BlockSpec((BT, 128), lambda i: (i, 0)),
        ],
        out_shape=[
            jax.ShapeDtypeStruct((T, D), _f32),
            jax.ShapeDtypeStruct((T, D), _f32),
            jax.ShapeDtypeStruct((T, D), _f32),
            jax.ShapeDtypeStruct((T, 128), _f32),
            jax.ShapeDtypeStruct((T, 128), jnp.int32),
        ],
        compiler_params=_CP,
    )(ao, res, w_o, wlnp2d, w_share_gu, w_share_down, wg_pad)


def _moe_body(eid_ref, xg, ws, wgu, wdn, h2g, p0g, p1g, eo):
    del eid_ref, h2g, p0g, p1g
    g = _dot(xg[...], wgu[0])
    g1 = g[:, :F]
    g2 = g[:, F:]
    act = g1 * jax.nn.sigmoid(g1) * g2
    act = act * ws[:, :1]
    eo[...] = _dot(act, wdn[0])


def _moe(eid, xg, ws2d, w_gu, w_down, h2, pos0_2d, pos1_2d):
    grid_spec = pltpu.PrefetchScalarGridSpec(
        num_scalar_prefetch=1,
        grid=(NTM,),
        in_specs=[
            pl.BlockSpec((BT, D), lambda i, eid: (i, 0)),
            pl.BlockSpec((BT, 128), lambda i, eid: (i, 0)),
            pl.BlockSpec((1, D, 2 * F), lambda i, eid: (eid[i], 0, 0)),
            pl.BlockSpec((1, F, D), lambda i, eid: (eid[i], 0, 0)),
            pl.BlockSpec((8, 128), lambda i, eid: (0, 0)),
            pl.BlockSpec((8, _CH2), lambda i, eid: (0, 0)),
            pl.BlockSpec((8, _CH2), lambda i, eid: (0, 0)),
        ],
        out_specs=pl.BlockSpec((BT, D), lambda i, eid: (i, 0)),
    )
    return pl.pallas_call(
        _moe_body,
        grid_spec=grid_spec,
        out_shape=jax.ShapeDtypeStruct((NPAD, D), _f32),
        compiler_params=_CP,
    )(eid, xg, ws2d, w_gu, w_down, h2, pos0_2d, pos1_2d)


_NC = 2
_NS = 16
_NW = _NC * _NS
_ROWS_W = NPAD // _NW
_CH = 24
_NCH = _ROWS_W // _CH


def _sc_mesh():
    return plsc.VectorSubcoreMesh(core_axis_name="c", subcore_axis_name="s")


def _sc_scratch():
    return [
        pltpu.VMEM((_NCH, _CH), jnp.int32),
        pltpu.VMEM((2, _CH, D), _f32),
        pltpu.SemaphoreType.DMA,
        pltpu.SemaphoreType.DMA,
        pltpu.SemaphoreType.DMA,
        pltpu.SemaphoreType.DMA,
    ]


def _sc_pipeline(nch, read_chunk, write_chunk, rows_v, gsems, wsems):
    gat = [None] * nch
    wrt = [None] * nch
    gat[0] = read_chunk(0, rows_v.at[0], gsems[0])
    for c in range(nch):
        if c + 1 < nch:
            if c - 1 >= 0:
                for d in wrt[c - 1]:
                    d.wait()
            gat[c + 1] = read_chunk(c + 1, rows_v.at[(c + 1) % 2],
                                    gsems[(c + 1) % 2])
        gat[c].wait()
        wrt[c] = write_chunk(c, rows_v.at[c % 2], wsems[c % 2])
    for c in (nch - 2, nch - 1):
        for d in wrt[c]:
            d.wait()


_TPW = T // _NW
_CH2 = 16
_NCH2 = _TPW // _CH2


def _dispatch_body(h2_hbm, pos0_hbm, pos1_hbm, out_hbm,
                   idx0_v, idx1_v, rows_v, s0, s1, s2, s3):
    wid = lax.axis_index("s") * _NC + lax.axis_index("c")
    base = wid * _TPW
    pltpu.sync_copy(pos0_hbm.at[pl.ds(wid * _NCH2, _NCH2)], idx0_v)
    pltpu.sync_copy(pos1_hbm.at[pl.ds(wid * _NCH2, _NCH2)], idx1_v)

    def read_chunk(c, buf, sem):
        return pltpu.async_copy(
            h2_hbm.at[pl.ds(base + c * _CH2, _CH2)], buf, sem)

    def write_chunk(c, buf, sem):
        return [pltpu.async_copy(buf, out_hbm.at[idx0_v.at[c]], sem),
                pltpu.async_copy(buf, out_hbm.at[idx1_v.at[c]], sem)]

    _sc_pipeline(_NCH2, read_chunk, write_chunk, rows_v, (s0, s1), (s2, s3))


def _dispatch(h2, pos0_2d, pos1_2d):
    return pl.kernel(
        _dispatch_body,
        out_type=jax.ShapeDtypeStruct((NPAD, D), _f32),
        mesh=_sc_mesh(),
        scratch_types=[
            pltpu.VMEM((_NCH2, _CH2), jnp.int32),
            pltpu.VMEM((_NCH2, _CH2), jnp.int32),
            pltpu.VMEM((2, _CH2, D), _f32),
            pltpu.SemaphoreType.DMA,
            pltpu.SemaphoreType.DMA,
            pltpu.SemaphoreType.DMA,
            pltpu.SemaphoreType.DMA,
        ],
    )(h2, pos0_2d, pos1_2d)


def _combine_body(eo_hbm, tgt_hbm, out_hbm, idx_v, rows_v, s0, s1, s2, s3):
    wid = lax.axis_index("s") * _NC + lax.axis_index("c")
    base = wid * _ROWS_W
    pltpu.sync_copy(tgt_hbm.at[pl.ds(wid * _NCH, _NCH)], idx_v)

    def read_chunk(c, buf, sem):
        return pltpu.async_copy(
            eo_hbm.at[pl.ds(base + c * _CH, _CH)], buf, sem)

    def write_chunk(c, buf, sem):
        return [pltpu.async_copy(buf, out_hbm.at[idx_v.at[c]], sem)]

    _sc_pipeline(_NCH, read_chunk, write_chunk, rows_v, (s0, s1), (s2, s3))


def _combine(eo, tgt2d):
    return pl.kernel(
        _combine_body,
        out_type=jax.ShapeDtypeStruct((2 * T + TRASH, D), _f32),
        mesh=_sc_mesh(),
        scratch_types=_sc_scratch(),
    )(eo, tgt2d)


def _final_body(share, b0, b1, eog, tgtg, out):
    del eog, tgtg
    out[...] = share[...] + b0[...] + b1[...]


def _final(share, buf, eo, tgt2d):
    return pl.pallas_call(
        _final_body,
        grid=(NT,),
        in_specs=[
            pl.BlockSpec((BT, D), lambda i: (i, 0)),
            pl.BlockSpec((BT, D), lambda i: (i, 0)),
            pl.BlockSpec((BT, D), lambda i: (i + NT, 0)),
            pl.BlockSpec((8, 128), lambda i: (0, 0)),
            pl.BlockSpec((8, _CH), lambda i: (0, 0)),
        ],
        out_specs=pl.BlockSpec((BT, D), lambda i: (i, 0)),
        out_shape=jax.ShapeDtypeStruct((T, D), _f32),
    )(share, buf, buf, eo, tgt2d)


def kernel(positions, hidden_states, residual, w_ln_in, w_qkv, w_inter,
           w_q, w_o, w_ln_post, w_gate, w_gu_experts, w_down_experts,
           w_share_gu, w_share_down):
    wqkv_f = w_ln_in[:, None] * w_qkv
    wq_f = w_inter[:, None] * w_q
    wlnp2d = w_ln_post.reshape(1, D)
    wg_pad = jnp.zeros((D, 128), _f32).at[:, :E].set(w_gate)
    inv = 1.0 / (10000.0 ** (jnp.arange(0, DH, 2, dtype=_f32) / DH))
    f = positions.astype(_f32)[:, None] * inv[None, :]
    cos = jnp.cos(f)
    sin = jnp.sin(f)
    ck = jnp.concatenate([cos, cos], axis=1)
    sk = jnp.concatenate([-sin, sin], axis=1)
    cq = jnp.tile(ck, (1, NH))
    sq = jnp.tile(sk, (1, NH))

    q, k, v, res = _prologue(hidden_states, residual, wqkv_f, wq_f,
                             cq, sq, ck, sk)
    ao = _attention(q, k, v)
    res2, h2, share, pv, pi = _post(ao, res, w_o, wlnp2d,
                                    w_share_gu, w_share_down, wg_pad)

    i32 = jnp.int32
    e_flat = jnp.stack([pi[:, 0], pi[:, 1]], axis=1).reshape(-1)
    w_flat = jnp.stack([pv[:, 0], pv[:, 1]], axis=1).reshape(-1)
    tok = jnp.arange(T, dtype=i32)
    tgt_tok = jnp.stack([tok, tok + T], axis=1).reshape(-1)
    oh = (e_flat[:, None] == jnp.arange(E, dtype=i32)[None, :]).astype(i32)
    counts = jnp.sum(oh, axis=0)
    rank = jnp.sum((jnp.cumsum(oh, axis=0) - oh) * oh, axis=1)
    cap = ((counts + BT - 1) // BT) * BT
    cum = jnp.cumsum(cap)
    off = cum - cap
    pos = off[e_flat] + rank
    ws = jnp.zeros((NPAD,), _f32).at[pos].set(w_flat)
    tgt = (2 * T + jnp.arange(NPAD, dtype=i32)).at[pos].set(tgt_tok)
    eid = jnp.minimum(
        jnp.sum(jnp.arange(NTM, dtype=i32)[:, None] * BT >= cum[None, :],
                axis=1), E - 1).astype(i32)

    pos2 = pos.reshape(T, 2)
    pos0_2d = pos2[:, 0].reshape(T // _CH2, _CH2)
    pos1_2d = pos2[:, 1].reshape(T // _CH2, _CH2)
    xg = _dispatch(h2, pos0_2d, pos1_2d)
    ws2d = jnp.broadcast_to(ws[:, None], (NPAD, 128))
    eo = _moe(eid, xg, ws2d, w_gu_experts, w_down_experts,
              h2, pos0_2d, pos1_2d)
    tgt2d = tgt.reshape(NPAD // _CH, _CH)
    buf = _combine(eo, tgt2d)
    out = _final(share, buf, eo, tgt2d)
    return out, res2
